# baseline (speedup 1.0000x reference)
import functools

import jax
import jax.numpy as jnp
import numpy as np
from jax.experimental import pallas as pl

_R = 4096
_N = 64
_STEPS = 4
_PER = 16
_BASE_VAR = 64.0
_HID = 128
_HIST_PAD = 1e-05
_EPS = 1e-05
_B = 64

_F32 = jnp.float32


def _cumsum(x):
    n = x.shape[-1]
    s = 1
    while s < n:
        x = x + jnp.concatenate(
            [jnp.zeros_like(x[..., :s]), x[..., :-s]], axis=-1)
        s *= 2
    return x


def _cumprod(x):
    n = x.shape[-1]
    s = 1
    while s < n:
        x = x * jnp.concatenate(
            [jnp.ones_like(x[..., :s]), x[..., :-s]], axis=-1)
        s *= 2
    return x


def _sigmoid(x):
    return 1.0 / (1.0 + jnp.exp(-x))


def _sdf_eval(o, d, t, W1, b1, W2, b2):
    Bq, K = t.shape
    starts = o[:, None, :] + t[:, :, None] * d[:, None, :]
    p = starts.reshape(Bq * K, 3)
    h = jnp.dot(p, W1, preferred_element_type=_F32) + b1[None, :]
    h = jnp.maximum(h, 0.0)
    s = jnp.dot(h, W2, preferred_element_type=_F32)
    return s.reshape(Bq, K) + b2


def _render_alpha(bins, max_bin, sdf, inv_s):
    edges = jnp.concatenate([bins, max_bin], axis=-1)
    dists = edges[:, 1:] - edges[:, :-1]
    dist = dists[:, :-1]
    prev_sdf = sdf[:, :-1]
    next_sdf = sdf[:, 1:]
    mid_sdf = (prev_sdf + next_sdf) * 0.5
    cos_val = (next_sdf - prev_sdf) / (dist + 1e-05)
    prev_cos = jnp.concatenate(
        [jnp.zeros_like(cos_val[:, :1]), cos_val[:, :-1]], axis=-1)
    cos_val = jnp.clip(jnp.minimum(prev_cos, cos_val), -1000.0, 0.0)
    prev_esti = mid_sdf - cos_val * dist * 0.5
    next_esti = mid_sdf + cos_val * dist * 0.5
    prev_cdf = _sigmoid(prev_esti * inv_s)
    next_cdf = _sigmoid(next_esti * inv_s)
    alpha = (prev_cdf - next_cdf + 1e-05) / (prev_cdf + 1e-05)
    return jnp.concatenate([alpha, jnp.zeros_like(alpha[:, :1])], axis=-1)


def _weights(alpha):
    ta = jnp.concatenate(
        [jnp.ones_like(alpha[:, :1]), 1.0 - alpha[:, :-1]], axis=-1)
    return alpha * _cumprod(ta)


def _pdf_sample(bins, max_bin, w, u3, u2):
    n = w.shape[-1]
    w = w + _HIST_PAD
    wsum = jnp.sum(w, axis=-1, keepdims=True)
    padding = jnp.maximum(_EPS - wsum, 0.0)
    w = w + padding / n
    pdf = w / (wsum + padding)
    cdf = jnp.minimum(_cumsum(pdf), 1.0)
    cdf = jnp.concatenate([jnp.zeros_like(cdf[:, :1]), cdf], axis=-1)
    existing = jnp.concatenate([bins, max_bin], axis=-1)

    M = cdf[:, None, :] <= u3
    cdf_b = cdf[:, None, :]
    ex_b = existing[:, None, :]
    cdf_g0 = jnp.max(jnp.where(M, cdf_b, 0.0), axis=-1)
    bins_g0 = jnp.max(jnp.where(M, ex_b, 0.0), axis=-1)
    cdf_g1 = jnp.min(jnp.where(M, 2.0, cdf_b), axis=-1)
    bins_g1 = jnp.min(jnp.where(M, jnp.float32(1e30), ex_b), axis=-1)
    all_m = M[:, :, -1]
    cdf_g1 = jnp.where(all_m, cdf[:, -1:], cdf_g1)
    bins_g1 = jnp.where(all_m, existing[:, -1:], bins_g1)

    denom = cdf_g1 - cdf_g0
    raw = (u2 - cdf_g0) / denom
    raw = jnp.where(jnp.isnan(raw), 0.0, raw)
    ts = jnp.clip(raw, 0.0, 1.0)
    return bins_g0 + ts * (bins_g1 - bins_g0)


def _merge_pos(bins, new16):
    Bq, n = bins.shape
    k = new16.shape[-1]
    Mlt = (new16[:, :, None] < bins[:, None, :]).astype(_F32)
    pos_old = (jax.lax.broadcasted_iota(jnp.int32, (Bq, n), 1).astype(_F32)
               + jnp.sum(Mlt, axis=1))
    pos_new = (jax.lax.broadcasted_iota(jnp.int32, (Bq, k), 1).astype(_F32)
               + (n - jnp.sum(Mlt, axis=2)))
    return pos_old, pos_new


def _scatter_merge(vals_old, vals_new, pos_old, pos_new):
    Bq, n = vals_old.shape
    k = vals_new.shape[-1]
    m = n + k
    c = pos_old - jax.lax.broadcasted_iota(jnp.int32, (Bq, n), 1).astype(_F32)
    pad_v = jnp.zeros((Bq, k), _F32)
    pad_c = jnp.full((Bq, k), -1.0, _F32)
    old_p = jnp.concatenate([vals_old, pad_v], axis=-1)
    c_p = jnp.concatenate([c, pad_c], axis=-1)
    acc = jnp.zeros((Bq, m), _F32)
    for s in range(k + 1):
        if s == 0:
            old_s, c_s = old_p, c_p
        else:
            old_s = jnp.concatenate(
                [jnp.zeros((Bq, s), _F32), old_p[:, :-s]], axis=-1)
            c_s = jnp.concatenate(
                [jnp.full((Bq, s), -1.0, _F32), c_p[:, :-s]], axis=-1)
        acc = acc + jnp.where(c_s == np.float32(s), old_s, 0.0)
    oh_new = (pos_new[:, :, None]
              == jax.lax.broadcasted_iota(jnp.int32, (Bq, k, m), 2).astype(_F32)).astype(_F32)
    return acc + jnp.sum(oh_new * vals_new[:, :, None], axis=1)


def _body(o_ref, d_ref, w1_ref, b1_ref, w2_ref, b2_ref, out_ref):
    o = o_ref[...]
    d = d_ref[...]
    W1 = w1_ref[...]
    b1 = b1_ref[0, :]
    W2 = w2_ref[...]
    b2 = b2_ref[0, 0]

    num_bins = _PER + 1
    du = np.float32(np.float32(1.0 - 1.0 / num_bins) / (num_bins - 1))
    u0 = np.float32(0.5 / num_bins)
    u3 = jax.lax.broadcasted_iota(jnp.int32, (1, num_bins, 1), 1).astype(_F32) * du + u0
    u2 = jax.lax.broadcasted_iota(jnp.int32, (1, num_bins), 1).astype(_F32) * du + u0

    bins = jax.lax.broadcasted_iota(jnp.int32, (_B, _N), 1).astype(_F32) * np.float32(1.0 / _N)
    max_bin = jnp.full((_B, 1), 1.0, dtype=_F32)
    new_t = bins
    sdf = None
    pos_old = pos_new = None

    for it in range(_STEPS):
        new_sdf = _sdf_eval(o, d, new_t, W1, b1, W2, b2)
        if sdf is None:
            sdf = new_sdf
        else:
            sdf = _scatter_merge(sdf, new_sdf, pos_old, pos_new)
        alpha = _render_alpha(bins, max_bin, sdf, _BASE_VAR * (2 ** it))
        w = _weights(alpha)
        nb = _pdf_sample(bins, max_bin, w, u3, u2)
        new16 = nb[:, :_PER]
        new_max = nb[:, _PER:]
        pos_old, pos_new = _merge_pos(bins, new16)
        bins = _scatter_merge(bins, new16, pos_old, pos_new)
        max_bin = jnp.maximum(max_bin, new_max)
        new_t = new16

    out_ref[...] = bins


@jax.jit
def kernel(origins, directions, W1, b1, W2, b2):
    out = pl.pallas_call(
        _body,
        grid=(_R // _B,),
        in_specs=[
            pl.BlockSpec((_B, 3), lambda i: (i, 0)),
            pl.BlockSpec((_B, 3), lambda i: (i, 0)),
            pl.BlockSpec((3, _HID), lambda i: (0, 0)),
            pl.BlockSpec((1, _HID), lambda i: (0, 0)),
            pl.BlockSpec((_HID, 1), lambda i: (0, 0)),
            pl.BlockSpec((1, 1), lambda i: (0, 0)),
        ],
        out_specs=pl.BlockSpec((_B, _N + _STEPS * _PER), lambda i: (i, 0)),
        out_shape=jax.ShapeDtypeStruct((_R, _N + _STEPS * _PER), _F32),
    )(origins, directions, W1, b1.reshape(1, _HID), W2, b2.reshape(1, 1))
    return out[..., None]


# speedup vs baseline: 1.1838x; 1.1838x over previous
import functools

import jax
import jax.numpy as jnp
import numpy as np
from jax.experimental import pallas as pl

_R = 4096
_N = 64
_STEPS = 4
_PER = 16
_BASE_VAR = 64.0
_HID = 128
_HIST_PAD = 1e-05
_EPS = 1e-05
_B = 128

_F32 = jnp.float32


def _cumsum(x):
    n = x.shape[-1]
    s = 1
    while s < n:
        x = x + jnp.concatenate(
            [jnp.zeros_like(x[..., :s]), x[..., :-s]], axis=-1)
        s *= 2
    return x


def _cumprod(x):
    n = x.shape[-1]
    s = 1
    while s < n:
        x = x * jnp.concatenate(
            [jnp.ones_like(x[..., :s]), x[..., :-s]], axis=-1)
        s *= 2
    return x


def _sigmoid(x):
    return 1.0 / (1.0 + jnp.exp(-x))


def _sdf_eval(o, d, t, W1, b1, W2, b2):
    Bq, K = t.shape
    starts = o[:, None, :] + t[:, :, None] * d[:, None, :]
    p = starts.reshape(Bq * K, 3)
    h = jnp.dot(p, W1, preferred_element_type=_F32) + b1[None, :]
    h = jnp.maximum(h, 0.0)
    s = jnp.dot(h, W2, preferred_element_type=_F32)
    return s.reshape(Bq, K) + b2


def _render_alpha(bins, max_bin, sdf, inv_s):
    edges = jnp.concatenate([bins, max_bin], axis=-1)
    dists = edges[:, 1:] - edges[:, :-1]
    dist = dists[:, :-1]
    prev_sdf = sdf[:, :-1]
    next_sdf = sdf[:, 1:]
    mid_sdf = (prev_sdf + next_sdf) * 0.5
    cos_val = (next_sdf - prev_sdf) / (dist + 1e-05)
    prev_cos = jnp.concatenate(
        [jnp.zeros_like(cos_val[:, :1]), cos_val[:, :-1]], axis=-1)
    cos_val = jnp.clip(jnp.minimum(prev_cos, cos_val), -1000.0, 0.0)
    prev_esti = mid_sdf - cos_val * dist * 0.5
    next_esti = mid_sdf + cos_val * dist * 0.5
    prev_cdf = _sigmoid(prev_esti * inv_s)
    next_cdf = _sigmoid(next_esti * inv_s)
    alpha = (prev_cdf - next_cdf + 1e-05) / (prev_cdf + 1e-05)
    return jnp.concatenate([alpha, jnp.zeros_like(alpha[:, :1])], axis=-1)


def _weights(alpha):
    ta = jnp.concatenate(
        [jnp.ones_like(alpha[:, :1]), 1.0 - alpha[:, :-1]], axis=-1)
    return alpha * _cumprod(ta)


def _pdf_sample(bins, max_bin, w, u3, u2):
    n = w.shape[-1]
    w = w + _HIST_PAD
    wsum = jnp.sum(w, axis=-1, keepdims=True)
    padding = jnp.maximum(_EPS - wsum, 0.0)
    w = w + padding / n
    pdf = w / (wsum + padding)
    cdf = jnp.minimum(_cumsum(pdf), 1.0)
    cdf = jnp.concatenate([jnp.zeros_like(cdf[:, :1]), cdf], axis=-1)
    existing = jnp.concatenate([bins, max_bin], axis=-1)

    M = cdf[:, None, :] <= u3
    cdf_b = cdf[:, None, :]
    ex_b = existing[:, None, :]
    cdf_g0 = jnp.max(jnp.where(M, cdf_b, 0.0), axis=-1)
    bins_g0 = jnp.max(jnp.where(M, ex_b, 0.0), axis=-1)
    cdf_g1 = jnp.min(jnp.where(M, 2.0, cdf_b), axis=-1)
    bins_g1 = jnp.min(jnp.where(M, jnp.float32(1e30), ex_b), axis=-1)
    all_m = M[:, :, -1]
    cdf_g1 = jnp.where(all_m, cdf[:, -1:], cdf_g1)
    bins_g1 = jnp.where(all_m, existing[:, -1:], bins_g1)

    denom = cdf_g1 - cdf_g0
    raw = (u2 - cdf_g0) / denom
    raw = jnp.where(jnp.isnan(raw), 0.0, raw)
    ts = jnp.clip(raw, 0.0, 1.0)
    return bins_g0 + ts * (bins_g1 - bins_g0)


def _merge_pos(bins, new16):
    Bq, n = bins.shape
    k = new16.shape[-1]
    Mlt = (new16[:, :, None] < bins[:, None, :]).astype(_F32)
    pos_old = (jax.lax.broadcasted_iota(jnp.int32, (Bq, n), 1).astype(_F32)
               + jnp.sum(Mlt, axis=1))
    pos_new = (jax.lax.broadcasted_iota(jnp.int32, (Bq, k), 1).astype(_F32)
               + (n - jnp.sum(Mlt, axis=2)))
    return pos_old, pos_new


def _scatter_merge(vals_old, vals_new, pos_old, pos_new):
    Bq, n = vals_old.shape
    k = vals_new.shape[-1]
    m = n + k
    c = pos_old - jax.lax.broadcasted_iota(jnp.int32, (Bq, n), 1).astype(_F32)
    pad_v = jnp.zeros((Bq, k), _F32)
    pad_c = jnp.full((Bq, k), -1.0, _F32)
    old_p = jnp.concatenate([vals_old, pad_v], axis=-1)
    c_p = jnp.concatenate([c, pad_c], axis=-1)
    acc = jnp.zeros((Bq, m), _F32)
    for s in range(k + 1):
        if s == 0:
            old_s, c_s = old_p, c_p
        else:
            old_s = jnp.concatenate(
                [jnp.zeros((Bq, s), _F32), old_p[:, :-s]], axis=-1)
            c_s = jnp.concatenate(
                [jnp.full((Bq, s), -1.0, _F32), c_p[:, :-s]], axis=-1)
        acc = acc + jnp.where(c_s == np.float32(s), old_s, 0.0)
    oh_new = (pos_new[:, :, None]
              == jax.lax.broadcasted_iota(jnp.int32, (Bq, k, m), 2).astype(_F32)).astype(_F32)
    return acc + jnp.sum(oh_new * vals_new[:, :, None], axis=1)


def _body(o_ref, d_ref, w1_ref, b1_ref, w2_ref, b2_ref, out_ref):
    o = o_ref[...]
    d = d_ref[...]
    W1 = w1_ref[...]
    b1 = b1_ref[0, :]
    W2 = w2_ref[...]
    b2 = b2_ref[0, 0]

    num_bins = _PER + 1
    du = np.float32(np.float32(1.0 - 1.0 / num_bins) / (num_bins - 1))
    u0 = np.float32(0.5 / num_bins)
    u3 = jax.lax.broadcasted_iota(jnp.int32, (1, num_bins, 1), 1).astype(_F32) * du + u0
    u2 = jax.lax.broadcasted_iota(jnp.int32, (1, num_bins), 1).astype(_F32) * du + u0

    bins = jax.lax.broadcasted_iota(jnp.int32, (_B, _N), 1).astype(_F32) * np.float32(1.0 / _N)
    max_bin = jnp.full((_B, 1), 1.0, dtype=_F32)
    new_t = bins
    sdf = None
    pos_old = pos_new = None

    for it in range(_STEPS):
        new_sdf = _sdf_eval(o, d, new_t, W1, b1, W2, b2)
        if sdf is None:
            sdf = new_sdf
        else:
            sdf = _scatter_merge(sdf, new_sdf, pos_old, pos_new)
        alpha = _render_alpha(bins, max_bin, sdf, _BASE_VAR * (2 ** it))
        w = _weights(alpha)
        nb = _pdf_sample(bins, max_bin, w, u3, u2)
        new16 = nb[:, :_PER]
        new_max = nb[:, _PER:]
        pos_old, pos_new = _merge_pos(bins, new16)
        bins = _scatter_merge(bins, new16, pos_old, pos_new)
        max_bin = jnp.maximum(max_bin, new_max)
        new_t = new16

    out_ref[...] = bins


@jax.jit
def kernel(origins, directions, W1, b1, W2, b2):
    out = pl.pallas_call(
        _body,
        grid=(_R // _B,),
        in_specs=[
            pl.BlockSpec((_B, 3), lambda i: (i, 0)),
            pl.BlockSpec((_B, 3), lambda i: (i, 0)),
            pl.BlockSpec((3, _HID), lambda i: (0, 0)),
            pl.BlockSpec((1, _HID), lambda i: (0, 0)),
            pl.BlockSpec((_HID, 1), lambda i: (0, 0)),
            pl.BlockSpec((1, 1), lambda i: (0, 0)),
        ],
        out_specs=pl.BlockSpec((_B, _N + _STEPS * _PER), lambda i: (i, 0)),
        out_shape=jax.ShapeDtypeStruct((_R, _N + _STEPS * _PER), _F32),
    )(origins, directions, W1, b1.reshape(1, _HID), W2, b2.reshape(1, 1))
    return out[..., None]
